# baseline (device time: 34695 ns/iter reference)
import jax
import jax.numpy as jnp
from jax import lax
from jax.experimental import pallas as pl
from jax.experimental.pallas import tpu as pltpu

B, S, H, D = 4, 512, 8, 64
HD = 1024
S_HALF = S // 2
ROWS = B * S_HALF


def kernel(O, Wo):
    def body(o_ref, wo_ref, out_ref, send_buf, recv_buf, send_sems, recv_sems):
        my_x = lax.axis_index("x")
        my_y = lax.axis_index("y")
        peer_x = 1 - my_x

        barrier = pltpu.get_barrier_semaphore()
        pl.semaphore_signal(
            barrier, inc=1,
            device_id=(peer_x, my_y), device_id_type=pl.DeviceIdType.MESH,
        )
        pl.semaphore_wait(barrier, 1)

        def partial_batch(b, s0):
            acc = jnp.zeros((S_HALF, HD), jnp.float32)
            for h in range(H):
                a = o_ref[b, pl.ds(s0, S_HALF), h, :]
                w = wo_ref[h * D:(h + 1) * D, :]
                acc += jnp.dot(a, w, preferred_element_type=jnp.float32)
            return acc

        def chunk_rdma(b):
            return pltpu.make_async_remote_copy(
                src_ref=send_buf.at[b],
                dst_ref=recv_buf.at[b],
                send_sem=send_sems.at[b],
                recv_sem=recv_sems.at[b],
                device_id=(peer_x, my_y),
                device_id_type=pl.DeviceIdType.MESH,
            )

        for b in range(B):
            send_buf[b] = partial_batch(b, peer_x * S_HALF).astype(jnp.bfloat16)
            chunk_rdma(b).start()

        for b in range(B):
            own = partial_batch(b, my_x * S_HALF)
            chunk_rdma(b).wait_recv()
            out_ref[b] = own + recv_buf[b].astype(jnp.float32)
        for b in range(B):
            chunk_rdma(b).wait_send()

    return pl.pallas_call(
        body,
        out_shape=jax.ShapeDtypeStruct((B, S_HALF, HD), jnp.float32),
        in_specs=[
            pl.BlockSpec(memory_space=pltpu.VMEM),
            pl.BlockSpec(memory_space=pltpu.VMEM),
        ],
        out_specs=pl.BlockSpec(memory_space=pltpu.VMEM),
        scratch_shapes=[
            pltpu.VMEM((B, S_HALF, HD), jnp.bfloat16),
            pltpu.VMEM((B, S_HALF, HD), jnp.bfloat16),
            pltpu.SemaphoreType.DMA((B,)),
            pltpu.SemaphoreType.DMA((B,)),
        ],
        compiler_params=pltpu.CompilerParams(collective_id=0),
    )(O, Wo)


# device time: 34144 ns/iter; 1.0161x vs baseline; 1.0161x over previous
import jax
import jax.numpy as jnp
from jax import lax
from jax.experimental import pallas as pl
from jax.experimental.pallas import tpu as pltpu

B, S, H, D = 4, 512, 8, 64
HD = 1024
S_HALF = S // 2
ROWS = B * S_HALF


def kernel(O, Wo):
    def body(o_ref, wo_ref, out_ref, send_buf, recv_buf, send_sems, recv_sems):
        my_x = lax.axis_index("x")
        my_y = lax.axis_index("y")
        peer_x = 1 - my_x

        barrier = pltpu.get_barrier_semaphore()
        pl.semaphore_signal(
            barrier, inc=1,
            device_id=(peer_x, my_y), device_id_type=pl.DeviceIdType.MESH,
        )
        pl.semaphore_wait(barrier, 1)

        NC = 2 * B
        CS = S_HALF // 2

        def partial_chunk(c, s0):
            acc = jnp.zeros((CS, HD), jnp.float32)
            for h in range(H):
                a = o_ref[c // 2, pl.ds(s0 + (c % 2) * CS, CS), h, :]
                w = wo_ref[h * D:(h + 1) * D, :]
                acc += jnp.dot(a, w, preferred_element_type=jnp.float32)
            return acc

        def chunk_rdma(c):
            return pltpu.make_async_remote_copy(
                src_ref=send_buf.at[c],
                dst_ref=recv_buf.at[c],
                send_sem=send_sems.at[c],
                recv_sem=recv_sems.at[c],
                device_id=(peer_x, my_y),
                device_id_type=pl.DeviceIdType.MESH,
            )

        for c in range(NC):
            send_buf[c] = partial_chunk(c, peer_x * S_HALF).astype(jnp.bfloat16)
            chunk_rdma(c).start()

        for c in range(NC):
            own = partial_chunk(c, my_x * S_HALF)
            chunk_rdma(c).wait_recv()
            out_ref[c // 2, (c % 2) * CS:(c % 2 + 1) * CS] = (
                own + recv_buf[c].astype(jnp.float32)
            )
        for c in range(NC):
            chunk_rdma(c).wait_send()

    return pl.pallas_call(
        body,
        out_shape=jax.ShapeDtypeStruct((B, S_HALF, HD), jnp.float32),
        in_specs=[
            pl.BlockSpec(memory_space=pltpu.VMEM),
            pl.BlockSpec(memory_space=pltpu.VMEM),
        ],
        out_specs=pl.BlockSpec(memory_space=pltpu.VMEM),
        scratch_shapes=[
            pltpu.VMEM((2 * B, S_HALF // 2, HD), jnp.bfloat16),
            pltpu.VMEM((2 * B, S_HALF // 2, HD), jnp.bfloat16),
            pltpu.SemaphoreType.DMA((2 * B,)),
            pltpu.SemaphoreType.DMA((2 * B,)),
        ],
        compiler_params=pltpu.CompilerParams(collective_id=0),
    )(O, Wo)
